# baseline (speedup 1.0000x reference)
import jax
import jax.numpy as jnp
from jax.experimental import pallas as pl
from jax.experimental.pallas import tpu as pltpu

_MT = 512
_FT = 256


def _router_body(x_ref, gd_ref, mask_ref, w0_ref, cm_ref):
    r = jnp.dot(x_ref[...], gd_ref[...], preferred_element_type=jnp.float32)
    w0 = jax.nn.sigmoid(r)
    w0_ref[...] = w0
    cm_ref[...] = mask_ref[...] / w0


def _moe_body(nf, s_tokens, x_ref, w1_ref, w3_ref, w2_ref, w0_ref, cm_ref,
              out_ref, hsum_ref):
    m = pl.program_id(0)
    ef = pl.program_id(1)
    e = ef // nf

    x = x_ref[...]
    h1 = jnp.dot(x, w1_ref[0], preferred_element_type=jnp.float32)
    h3 = jnp.dot(x, w3_ref[0], preferred_element_type=jnp.float32)
    w0 = w0_ref[...][:, 0:1]
    w = jnp.where(e == 0, w0, 1.0 - w0)
    hbig = jax.nn.silu(h1) * h3 * w
    hw = hbig

    part = jnp.dot(hw, w2_ref[0], preferred_element_type=jnp.float32)
    out_ref[...] = jnp.where(ef == 0, part, out_ref[...] + part)

    @pl.when((m == 0) & (ef == 0))
    def _init():
        hsum_ref[...] = jnp.zeros_like(hsum_ref)

    @pl.when(e == 0)
    def _cls():
        msum = jnp.sum(hbig * cm_ref[...][:, 0:1], axis=0, keepdims=True)
        b = (m * _MT) // s_tokens
        rows = jax.lax.broadcasted_iota(jnp.int32, (8, 1), 0)
        sl = pl.ds((ef % nf) * _FT, _FT)
        hsum_ref[:, sl] += jnp.where(rows == b, msum, 0.0)


def _cls_body(hsum_ref, w2_ref, mask_ref, clsw_ref, out_ref):
    ssum = jnp.dot(hsum_ref[...], w2_ref[...],
                   preferred_element_type=jnp.float32)
    lens = jnp.sum(mask_ref[...], axis=1, keepdims=True)
    avg = ssum / lens
    logits = jnp.dot(avg, clsw_ref[...],
                     preferred_element_type=jnp.float32)
    cols = jax.lax.broadcasted_iota(jnp.int32, logits.shape, 1)
    logits = jnp.where(cols < 7, logits, -1e30)
    mx = jnp.max(logits, axis=1, keepdims=True)
    p = jnp.exp(logits - mx)
    out_ref[...] = p / jnp.sum(p, axis=1, keepdims=True)


def kernel(hidden_states, attention_mask, task_id, gate_W, cls_W, W1, W3, W2):
    B, S, D = hidden_states.shape
    F = W1.shape[-1]
    M = B * S
    nm = M // _MT
    nf = F // _FT

    x = hidden_states.reshape(M, D)
    gd = jnp.broadcast_to(gate_W[:, 0:1] - gate_W[:, 1:2], (D, 128))
    w1 = W1
    w3 = W3
    w2 = W2
    mask_cols = jnp.broadcast_to(attention_mask.reshape(M, 1), (M, 128))

    w0_col, cm_col = pl.pallas_call(
        _router_body,
        grid=(nm,),
        in_specs=[
            pl.BlockSpec((_MT, D), lambda m: (m, 0)),
            pl.BlockSpec((D, 128), lambda m: (0, 0)),
            pl.BlockSpec((_MT, 128), lambda m: (m, 0)),
        ],
        out_specs=[
            pl.BlockSpec((_MT, 128), lambda m: (m, 0)),
            pl.BlockSpec((_MT, 128), lambda m: (m, 0)),
        ],
        out_shape=[
            jax.ShapeDtypeStruct((M, 128), jnp.float32),
            jax.ShapeDtypeStruct((M, 128), jnp.float32),
        ],
    )(x, gd, mask_cols)

    out, hsum = pl.pallas_call(
        lambda *refs: _moe_body(nf, S, *refs),
        grid=(nm, 2 * nf),
        in_specs=[
            pl.BlockSpec((_MT, D), lambda m, ef: (m, 0)),
            pl.BlockSpec((1, D, _FT), lambda m, ef: (ef // nf, 0, ef % nf)),
            pl.BlockSpec((1, D, _FT), lambda m, ef: (ef // nf, 0, ef % nf)),
            pl.BlockSpec((1, _FT, D), lambda m, ef: (ef // nf, ef % nf, 0)),
            pl.BlockSpec((_MT, 128), lambda m, ef: (m, 0)),
            pl.BlockSpec((_MT, 128), lambda m, ef: (m, 0)),
        ],
        out_specs=[
            pl.BlockSpec((_MT, D), lambda m, ef: (m, 0)),
            pl.BlockSpec((8, F), lambda m, ef: (0, 0)),
        ],
        out_shape=[
            jax.ShapeDtypeStruct((M, D), jnp.float32),
            jax.ShapeDtypeStruct((8, F), jnp.float32),
        ],
    )(x, w1, w3, w2, w0_col, cm_col)

    mask8 = jnp.concatenate([attention_mask, jnp.ones((8 - B, S), jnp.float32)], axis=0)
    clsw = jnp.pad(cls_W, ((0, 0), (0, 128 - cls_W.shape[1])))

    probs8 = pl.pallas_call(
        _cls_body,
        out_shape=jax.ShapeDtypeStruct((8, 128), jnp.float32),
    )(hsum, w2[0], mask8, clsw)

    return out.reshape(B, S, D), probs8[:B, :7]


# speedup vs baseline: 1.3212x; 1.3212x over previous
import jax
import jax.numpy as jnp
from jax.experimental import pallas as pl
from jax.experimental.pallas import tpu as pltpu

_MT = 1024
_FT = 256


def _router_body(x_ref, gd_ref, mask_ref, w0_ref, cm_ref):
    r = jnp.dot(x_ref[...], gd_ref[...], preferred_element_type=jnp.float32)
    w0 = jax.nn.sigmoid(r)
    w0_ref[...] = w0
    cm_ref[...] = mask_ref[...] / w0


def _moe_body(nf, s_tokens, x_ref, w1_ref, w3_ref, w2_ref, w0_ref, cm_ref,
              out_ref, hsum_ref):
    m = pl.program_id(0)
    ef = pl.program_id(1)
    e = ef // nf

    x = x_ref[...]
    h1 = jnp.dot(x, w1_ref[0], preferred_element_type=jnp.float32)
    h3 = jnp.dot(x, w3_ref[0], preferred_element_type=jnp.float32)
    w0 = w0_ref[...][:, 0:1]
    w = jnp.where(e == 0, w0, 1.0 - w0)
    hbig = jax.nn.silu(h1) * h3 * w
    hw = hbig

    part = jnp.dot(hw, w2_ref[0], preferred_element_type=jnp.float32)
    out_ref[...] = jnp.where(ef == 0, part, out_ref[...] + part)

    @pl.when((m == 0) & (ef == 0))
    def _init():
        hsum_ref[...] = jnp.zeros_like(hsum_ref)

    @pl.when(e == 0)
    def _cls():
        msum8 = jnp.dot(cm_ref[...], hbig, preferred_element_type=jnp.float32)
        sl = pl.ds((ef % nf) * _FT, _FT)
        hsum_ref[:, sl] += msum8


def _cls_body(hsum_ref, w2_ref, mask_ref, clsw_ref, out_ref):
    ssum = jnp.dot(hsum_ref[...], w2_ref[...],
                   preferred_element_type=jnp.float32)
    lens = jnp.sum(mask_ref[...], axis=1, keepdims=True)
    avg = ssum / lens
    logits = jnp.dot(avg, clsw_ref[...],
                     preferred_element_type=jnp.float32)
    cols = jax.lax.broadcasted_iota(jnp.int32, logits.shape, 1)
    logits = jnp.where(cols < 7, logits, -1e30)
    mx = jnp.max(logits, axis=1, keepdims=True)
    p = jnp.exp(logits - mx)
    out_ref[...] = p / jnp.sum(p, axis=1, keepdims=True)


def kernel(hidden_states, attention_mask, task_id, gate_W, cls_W, W1, W3, W2):
    B, S, D = hidden_states.shape
    F = W1.shape[-1]
    M = B * S
    nm = M // _MT
    nf = F // _FT

    x = hidden_states.reshape(M, D)
    gd = jnp.broadcast_to(gate_W[:, 0:1] - gate_W[:, 1:2], (D, 128))
    w1 = W1
    w3 = W3
    w2 = W2
    mask_cols = jnp.broadcast_to(attention_mask.reshape(M, 1), (M, 128))

    w0_col, cm_col = pl.pallas_call(
        _router_body,
        grid=(nm,),
        in_specs=[
            pl.BlockSpec((_MT, D), lambda m: (m, 0)),
            pl.BlockSpec((D, 128), lambda m: (0, 0)),
            pl.BlockSpec((_MT, 128), lambda m: (m, 0)),
        ],
        out_specs=[
            pl.BlockSpec((_MT, 128), lambda m: (m, 0)),
            pl.BlockSpec((_MT, 128), lambda m: (m, 0)),
        ],
        out_shape=[
            jax.ShapeDtypeStruct((M, 128), jnp.float32),
            jax.ShapeDtypeStruct((M, 128), jnp.float32),
        ],
    )(x, gd, mask_cols)

    batch_rows = jnp.arange(8)[:, None] == (jnp.arange(M) // S)[None, :]
    cmT8 = jnp.where(batch_rows, cm_col[:, 0][None, :], 0.0)

    out, hsum = pl.pallas_call(
        lambda *refs: _moe_body(nf, S, *refs),
        grid=(nm, 2 * nf),
        in_specs=[
            pl.BlockSpec((_MT, D), lambda m, ef: (m, 0)),
            pl.BlockSpec((1, D, _FT), lambda m, ef: (ef // nf, 0, ef % nf)),
            pl.BlockSpec((1, D, _FT), lambda m, ef: (ef // nf, 0, ef % nf)),
            pl.BlockSpec((1, _FT, D), lambda m, ef: (ef // nf, ef % nf, 0)),
            pl.BlockSpec((_MT, 128), lambda m, ef: (m, 0)),
            pl.BlockSpec((8, _MT), lambda m, ef: (0, m)),
        ],
        out_specs=[
            pl.BlockSpec((_MT, D), lambda m, ef: (m, 0)),
            pl.BlockSpec((8, F), lambda m, ef: (0, 0)),
        ],
        out_shape=[
            jax.ShapeDtypeStruct((M, D), jnp.float32),
            jax.ShapeDtypeStruct((8, F), jnp.float32),
        ],
    )(x, w1, w3, w2, w0_col, cmT8)

    mask8 = jnp.concatenate([attention_mask, jnp.ones((8 - B, S), jnp.float32)], axis=0)
    clsw = jnp.pad(cls_W, ((0, 0), (0, 128 - cls_W.shape[1])))

    probs8 = pl.pallas_call(
        _cls_body,
        out_shape=jax.ShapeDtypeStruct((8, 128), jnp.float32),
    )(hsum, w2[0], mask8, clsw)

    return out.reshape(B, S, D), probs8[:B, :7]


# speedup vs baseline: 1.4286x; 1.0813x over previous
import jax
import jax.numpy as jnp
from jax.experimental import pallas as pl
from jax.experimental.pallas import tpu as pltpu

_MT = 1024
_FT = 256


def _moe_body(nf, nm, x_ref, w1_ref, w3_ref, w2_ref, gd_ref, mt8_ref,
              out_ref, cls_ref, w0_scr, hsum_scr):
    m = pl.program_id(0)
    ef = pl.program_id(1)
    e = ef // nf

    @pl.when(ef == 0)
    def _router():
        r = jnp.dot(x_ref[...], gd_ref[...], preferred_element_type=jnp.float32)
        w0_scr[...] = jax.nn.sigmoid(r)

    @pl.when((m == 0) & (ef == 0))
    def _init():
        hsum_scr[...] = jnp.zeros_like(hsum_scr)
        cls_ref[...] = jnp.zeros_like(cls_ref)

    x = x_ref[...]
    h1 = jnp.dot(x, w1_ref[0], preferred_element_type=jnp.float32)
    h3 = jnp.dot(x, w3_ref[0], preferred_element_type=jnp.float32)
    h0 = jax.nn.silu(h1) * h3
    w0 = w0_scr[...][:, 0:1]
    w = jnp.where(e == 0, w0, 1.0 - w0)
    hw = h0 * w
    part = jnp.dot(hw, w2_ref[0], preferred_element_type=jnp.float32)
    out_ref[...] = jnp.where(ef == 0, part, out_ref[...] + part)

    @pl.when(e == 0)
    def _cls():
        sl = pl.ds((ef % nf) * _FT, _FT)
        hsum_scr[:, sl] += jnp.dot(mt8_ref[...], h0,
                                   preferred_element_type=jnp.float32)

        @pl.when(m == nm - 1)
        def _cls_fold():
            cls_ref[...] += jnp.dot(hsum_scr[:, sl], w2_ref[0],
                                    preferred_element_type=jnp.float32)


def _cls_body(cls_ref, mask_ref, clsw_ref, out_ref):
    lens = jnp.sum(mask_ref[...], axis=1, keepdims=True)
    avg = cls_ref[...] / lens
    logits = jnp.dot(avg, clsw_ref[...], preferred_element_type=jnp.float32)
    cols = jax.lax.broadcasted_iota(jnp.int32, logits.shape, 1)
    logits = jnp.where(cols < 7, logits, -1e30)
    mx = jnp.max(logits, axis=1, keepdims=True)
    p = jnp.exp(logits - mx)
    out_ref[...] = p / jnp.sum(p, axis=1, keepdims=True)


def kernel(hidden_states, attention_mask, task_id, gate_W, cls_W, W1, W3, W2):
    B, S, D = hidden_states.shape
    F = W1.shape[-1]
    M = B * S
    nm = M // _MT
    nf = F // _FT

    x = hidden_states.reshape(M, D)
    gd = jnp.broadcast_to(gate_W[:, 0:1] - gate_W[:, 1:2], (D, 128))
    batch_rows = jnp.arange(8)[:, None] == (jnp.arange(M) // S)[None, :]
    maskT8 = jnp.where(batch_rows, attention_mask.reshape(1, M), 0.0)

    out, cls_sum = pl.pallas_call(
        lambda *refs: _moe_body(nf, nm, *refs),
        grid=(nm, 2 * nf),
        in_specs=[
            pl.BlockSpec((_MT, D), lambda m, ef: (m, 0)),
            pl.BlockSpec((1, D, _FT), lambda m, ef: (ef // nf, 0, ef % nf)),
            pl.BlockSpec((1, D, _FT), lambda m, ef: (ef // nf, 0, ef % nf)),
            pl.BlockSpec((1, _FT, D), lambda m, ef: (ef // nf, ef % nf, 0)),
            pl.BlockSpec((D, 128), lambda m, ef: (0, 0)),
            pl.BlockSpec((8, _MT), lambda m, ef: (0, m)),
        ],
        out_specs=[
            pl.BlockSpec((_MT, D), lambda m, ef: (m, 0)),
            pl.BlockSpec((8, D), lambda m, ef: (0, 0)),
        ],
        out_shape=[
            jax.ShapeDtypeStruct((M, D), jnp.float32),
            jax.ShapeDtypeStruct((8, D), jnp.float32),
        ],
        scratch_shapes=[
            pltpu.VMEM((_MT, 128), jnp.float32),
            pltpu.VMEM((8, F), jnp.float32),
        ],
    )(x, W1, W3, W2, gd, maskT8)

    mask8 = jnp.concatenate([attention_mask, jnp.ones((8 - B, S), jnp.float32)], axis=0)
    clsw = jnp.pad(cls_W, ((0, 0), (0, 128 - cls_W.shape[1])))

    probs8 = pl.pallas_call(
        _cls_body,
        out_shape=jax.ShapeDtypeStruct((8, 128), jnp.float32),
    )(cls_sum, mask8, clsw)

    return out.reshape(B, S, D), probs8[:B, :7]


# speedup vs baseline: 1.4842x; 1.0389x over previous
import jax
import jax.numpy as jnp
from jax.experimental import pallas as pl
from jax.experimental.pallas import tpu as pltpu

_MT = 1024
_FT = 512


def _moe_body(nf, nm, x_ref, w1_ref, w3_ref, w2_ref, gd_ref, mt8_ref,
              out_ref, cls_ref, w0_scr, hsum_scr):
    m = pl.program_id(0)
    ef = pl.program_id(1)
    e = ef // nf

    @pl.when(ef == 0)
    def _router():
        r = jnp.dot(x_ref[...], gd_ref[...], preferred_element_type=jnp.float32)
        w0_scr[...] = jax.nn.sigmoid(r)

    @pl.when((m == 0) & (ef == 0))
    def _init():
        hsum_scr[...] = jnp.zeros_like(hsum_scr)
        cls_ref[...] = jnp.zeros_like(cls_ref)

    x = x_ref[...]
    h1 = jnp.dot(x, w1_ref[0], preferred_element_type=jnp.float32)
    h3 = jnp.dot(x, w3_ref[0], preferred_element_type=jnp.float32)
    h0 = jax.nn.silu(h1) * h3
    w0 = w0_scr[...][:, 0:1]
    w = jnp.where(e == 0, w0, 1.0 - w0)
    hw = h0 * w
    part = jnp.dot(hw, w2_ref[0], preferred_element_type=jnp.float32)
    out_ref[...] = jnp.where(ef == 0, part, out_ref[...] + part)

    @pl.when(e == 0)
    def _cls():
        sl = pl.ds((ef % nf) * _FT, _FT)
        hsum_scr[:, sl] += jnp.dot(mt8_ref[...], h0,
                                   preferred_element_type=jnp.float32)

        @pl.when(m == nm - 1)
        def _cls_fold():
            cls_ref[...] += jnp.dot(hsum_scr[:, sl], w2_ref[0],
                                    preferred_element_type=jnp.float32)


def _cls_body(cls_ref, mask_ref, clsw_ref, out_ref):
    lens = jnp.sum(mask_ref[...], axis=1, keepdims=True)
    avg = cls_ref[...] / lens
    logits = jnp.dot(avg, clsw_ref[...], preferred_element_type=jnp.float32)
    cols = jax.lax.broadcasted_iota(jnp.int32, logits.shape, 1)
    logits = jnp.where(cols < 7, logits, -1e30)
    mx = jnp.max(logits, axis=1, keepdims=True)
    p = jnp.exp(logits - mx)
    out_ref[...] = p / jnp.sum(p, axis=1, keepdims=True)


def kernel(hidden_states, attention_mask, task_id, gate_W, cls_W, W1, W3, W2):
    B, S, D = hidden_states.shape
    F = W1.shape[-1]
    M = B * S
    nm = M // _MT
    nf = F // _FT

    x = hidden_states.reshape(M, D)
    gd = jnp.broadcast_to(gate_W[:, 0:1] - gate_W[:, 1:2], (D, 128))
    batch_rows = jnp.arange(8)[:, None] == (jnp.arange(M) // S)[None, :]
    maskT8 = jnp.where(batch_rows, attention_mask.reshape(1, M), 0.0)

    out, cls_sum = pl.pallas_call(
        lambda *refs: _moe_body(nf, nm, *refs),
        grid=(nm, 2 * nf),
        in_specs=[
            pl.BlockSpec((_MT, D), lambda m, ef: (m, 0)),
            pl.BlockSpec((1, D, _FT), lambda m, ef: (ef // nf, 0, ef % nf)),
            pl.BlockSpec((1, D, _FT), lambda m, ef: (ef // nf, 0, ef % nf)),
            pl.BlockSpec((1, _FT, D), lambda m, ef: (ef // nf, ef % nf, 0)),
            pl.BlockSpec((D, 128), lambda m, ef: (0, 0)),
            pl.BlockSpec((8, _MT), lambda m, ef: (0, m)),
        ],
        out_specs=[
            pl.BlockSpec((_MT, D), lambda m, ef: (m, 0)),
            pl.BlockSpec((8, D), lambda m, ef: (0, 0)),
        ],
        out_shape=[
            jax.ShapeDtypeStruct((M, D), jnp.float32),
            jax.ShapeDtypeStruct((8, D), jnp.float32),
        ],
        scratch_shapes=[
            pltpu.VMEM((_MT, 128), jnp.float32),
            pltpu.VMEM((8, F), jnp.float32),
        ],
        compiler_params=pltpu.CompilerParams(
            vmem_limit_bytes=64 * 1024 * 1024,
        ),
    )(x, W1, W3, W2, gd, maskT8)

    mask8 = jnp.concatenate([attention_mask, jnp.ones((8 - B, S), jnp.float32)], axis=0)
    clsw = jnp.pad(cls_W, ((0, 0), (0, 128 - cls_W.shape[1])))

    probs8 = pl.pallas_call(
        _cls_body,
        out_shape=jax.ShapeDtypeStruct((8, 128), jnp.float32),
    )(cls_sum, mask8, clsw)

    return out.reshape(B, S, D), probs8[:B, :7]


# speedup vs baseline: 1.4916x; 1.0050x over previous
import jax
import jax.numpy as jnp
from jax.experimental import pallas as pl
from jax.experimental.pallas import tpu as pltpu

_MT = 1024
_FT = 512


def _moe_body(nf, nm, x_ref, w1_ref, w3_ref, w2_ref, gd_ref, mt8_ref,
              out_ref, cls_ref, w0_scr, hsum_scr):
    m = pl.program_id(0)
    ef = pl.program_id(1)
    e = ef // nf

    @pl.when(ef == 0)
    def _router():
        r = jnp.dot(x_ref[...], gd_ref[...], preferred_element_type=jnp.float32)
        w0_scr[...] = jax.nn.sigmoid(r)

    @pl.when((m == 0) & (ef == 0))
    def _init():
        hsum_scr[...] = jnp.zeros_like(hsum_scr)
        cls_ref[...] = jnp.zeros_like(cls_ref)

    x = x_ref[...]
    h1 = jnp.dot(x, w1_ref[0], preferred_element_type=jnp.float32)
    h3 = jnp.dot(x, w3_ref[0], preferred_element_type=jnp.float32)
    h0 = jax.nn.silu(h1) * h3
    w0 = w0_scr[...][:, 0:1]
    w = jnp.where(e == 0, w0, 1.0 - w0)
    hw = (h0 * w).astype(jnp.bfloat16)
    part = jnp.dot(hw, w2_ref[0], preferred_element_type=jnp.float32)
    out_ref[...] = jnp.where(ef == 0, part, out_ref[...] + part)

    @pl.when(e == 0)
    def _cls():
        sl = pl.ds((ef % nf) * _FT, _FT)
        hsum_scr[:, sl] += jnp.dot(mt8_ref[...], h0,
                                   preferred_element_type=jnp.float32)

        @pl.when(m == nm - 1)
        def _cls_fold():
            cls_ref[...] += jnp.dot(hsum_scr[:, sl], w2_ref[0],
                                    preferred_element_type=jnp.float32)


def _cls_body(cls_ref, mask_ref, clsw_ref, out_ref):
    lens = jnp.sum(mask_ref[...], axis=1, keepdims=True)
    avg = cls_ref[...] / lens
    logits = jnp.dot(avg, clsw_ref[...], preferred_element_type=jnp.float32)
    cols = jax.lax.broadcasted_iota(jnp.int32, logits.shape, 1)
    logits = jnp.where(cols < 7, logits, -1e30)
    mx = jnp.max(logits, axis=1, keepdims=True)
    p = jnp.exp(logits - mx)
    out_ref[...] = p / jnp.sum(p, axis=1, keepdims=True)


def kernel(hidden_states, attention_mask, task_id, gate_W, cls_W, W1, W3, W2):
    B, S, D = hidden_states.shape
    F = W1.shape[-1]
    M = B * S
    nm = M // _MT
    nf = F // _FT

    x = hidden_states.reshape(M, D)
    gd = jnp.broadcast_to(gate_W[:, 0:1] - gate_W[:, 1:2], (D, 128))
    batch_rows = jnp.arange(8)[:, None] == (jnp.arange(M) // S)[None, :]
    maskT8 = jnp.where(batch_rows, attention_mask.reshape(1, M), 0.0)

    out, cls_sum = pl.pallas_call(
        lambda *refs: _moe_body(nf, nm, *refs),
        grid=(nm, 2 * nf),
        in_specs=[
            pl.BlockSpec((_MT, D), lambda m, ef: (m, 0)),
            pl.BlockSpec((1, D, _FT), lambda m, ef: (ef // nf, 0, ef % nf)),
            pl.BlockSpec((1, D, _FT), lambda m, ef: (ef // nf, 0, ef % nf)),
            pl.BlockSpec((1, _FT, D), lambda m, ef: (ef // nf, ef % nf, 0)),
            pl.BlockSpec((D, 128), lambda m, ef: (0, 0)),
            pl.BlockSpec((8, _MT), lambda m, ef: (0, m)),
        ],
        out_specs=[
            pl.BlockSpec((_MT, D), lambda m, ef: (m, 0)),
            pl.BlockSpec((8, D), lambda m, ef: (0, 0)),
        ],
        out_shape=[
            jax.ShapeDtypeStruct((M, D), jnp.float32),
            jax.ShapeDtypeStruct((8, D), jnp.float32),
        ],
        scratch_shapes=[
            pltpu.VMEM((_MT, 128), jnp.float32),
            pltpu.VMEM((8, F), jnp.float32),
        ],
        compiler_params=pltpu.CompilerParams(
            vmem_limit_bytes=64 * 1024 * 1024,
        ),
    )(x, W1, W3, W2, gd, maskT8)

    mask8 = jnp.concatenate([attention_mask, jnp.ones((8 - B, S), jnp.float32)], axis=0)
    clsw = jnp.pad(cls_W, ((0, 0), (0, 128 - cls_W.shape[1])))

    probs8 = pl.pallas_call(
        _cls_body,
        out_shape=jax.ShapeDtypeStruct((8, 128), jnp.float32),
    )(cls_sum, mask8, clsw)

    return out.reshape(B, S, D), probs8[:B, :7]
